# Initial kernel scaffold; baseline (speedup 1.0000x reference)
import jax
import jax.numpy as jnp
from jax.experimental import pallas as pl


def kernel(x, coefficients_vect, scaling, knots):
    raise NotImplementedError("write your pallas kernel here")


# speedup vs baseline: 4369.7388x; 4369.7388x over previous
import functools

import jax
import jax.numpy as jnp
from jax import lax
from jax.experimental import pallas as pl
from jax.experimental.pallas import tpu as pltpu
from jax.experimental.pallas import tpu_sc as plsc

NUM_ACT = 4096
SIZE = 256
BATCH = 4096

NC = 2
NS = 16
L = 16
NW = NC * NS
P_PER_W = NUM_ACT // NW
QC = 256
NCHUNK = BATCH // QC
NQV = QC // L


def _make_sc_kernel():
    mesh = plsc.VectorSubcoreMesh(core_axis_name="c", subcore_axis_name="s")

    @functools.partial(
        pl.kernel,
        out_type=jax.ShapeDtypeStruct((NUM_ACT, BATCH), jnp.float32),
        mesh=mesh,
        compiler_params=pltpu.CompilerParams(
            use_tc_tiling_on_sc=False, needs_layout_passes=False),
        scratch_types=[
            pltpu.VMEM((QC, P_PER_W), jnp.float32),
            pltpu.VMEM((P_PER_W, SIZE), jnp.float32),
            pltpu.VMEM((BATCH,), jnp.float32),
            pltpu.VMEM((SIZE,), jnp.float32),
            pltpu.VMEM((3, L), jnp.float32),
            pltpu.VMEM((P_PER_W, QC), jnp.float32),
        ],
    )
    def k(x_hbm, coef_hbm, scal_hbm, knots_hbm, par_hbm, out_hbm,
          xv, cv, sv, kv, pv, ov):
        wid = lax.axis_index("s") * NC + lax.axis_index("c")
        p0 = wid * P_PER_W

        pltpu.sync_copy(coef_hbm.at[pl.ds(p0, P_PER_W), :], cv)
        pltpu.sync_copy(scal_hbm, sv)
        pltpu.sync_copy(knots_hbm, kv)
        pltpu.sync_copy(par_hbm, pv)

        vlo = pv[0]
        vh = pv[1]
        vinv_h = pv[2]
        viota = lax.iota(jnp.int32, L)
        vone = jnp.full((L,), 1.0, jnp.float32)
        kmax = jnp.full((L,), SIZE - 2, jnp.int32)
        kzero = jnp.full((L,), 0, jnp.int32)

        def chunk_body(qc_i, _):
            q0 = qc_i * QC
            pltpu.sync_copy(x_hbm.at[pl.ds(q0, QC), pl.ds(p0, P_PER_W)], xv)

            def qv_body(qv_i, _):
                qrow = viota + qv_i * L
                svec = sv[pl.ds(q0 + qv_i * L, L)]

                def p_body(p_i, _):
                    pfull = jnp.full((L,), p_i, jnp.int32)
                    xvec = plsc.load_gather(xv, [qrow, pfull])
                    u = (xvec - vlo) * vinv_h
                    est = jnp.clip(u.astype(jnp.int32), kzero, kmax)
                    a = plsc.load_gather(kv, [est])
                    b = plsc.load_gather(kv, [est + 1])
                    adj = jnp.where(xvec > b, 1, jnp.where(xvec <= a, -1, 0))
                    left = jnp.clip(est + adj, kzero, kmax)
                    lv = vlo + left.astype(jnp.float32) * vh
                    t = (xvec - lv) * vinv_h
                    cl = plsc.load_gather(cv, [pfull, left])
                    cr = plsc.load_gather(cv, [pfull, left + 1])
                    r = (cl * t + cr * (vone - t)) * svec
                    ov[p_i, pl.ds(qv_i * L, L)] = r
                    return 0

                lax.fori_loop(0, P_PER_W, p_body, 0)
                return 0

            lax.fori_loop(0, NQV, qv_body, 0)
            pltpu.sync_copy(ov, out_hbm.at[pl.ds(p0, P_PER_W), pl.ds(q0, QC)])
            return 0

        lax.fori_loop(0, NCHUNK, chunk_body, 0)

    return k


_sc_spline = _make_sc_kernel()


def kernel(x, coefficients_vect, scaling, knots):
    x2 = x.reshape(BATCH, NUM_ACT)
    coef2 = coefficients_vect.reshape(NUM_ACT, SIZE)
    scal1 = scaling.reshape(NUM_ACT)
    krow = knots[0]
    lo = krow[0]
    h = (krow[SIZE - 1] - krow[0]) / jnp.float32(SIZE - 1)
    inv_h = jnp.float32(1.0) / h
    params = jnp.stack([
        jnp.full((L,), lo, jnp.float32),
        jnp.full((L,), h, jnp.float32),
        jnp.full((L,), inv_h, jnp.float32),
    ])
    out2 = _sc_spline(x2, coef2, scal1, krow, params)
    return out2.reshape(x.shape)


# speedup vs baseline: 12369.8249x; 2.8308x over previous
import functools

import jax
import jax.numpy as jnp
from jax import lax
from jax.experimental import pallas as pl
from jax.experimental.pallas import tpu as pltpu
from jax.experimental.pallas import tpu_sc as plsc

NUM_ACT = 4096
SIZE = 256
BATCH = 4096

NC = 2
NS = 16
L = 16
NW = NC * NS
P_PER_W = NUM_ACT // NW
QC = 256
NCHUNK = BATCH // QC
NQV = QC // L


def _make_sc_kernel():
    mesh = plsc.VectorSubcoreMesh(core_axis_name="c", subcore_axis_name="s")

    @functools.partial(
        pl.kernel,
        out_type=jax.ShapeDtypeStruct((NUM_ACT, BATCH), jnp.float32),
        mesh=mesh,
        compiler_params=pltpu.CompilerParams(
            use_tc_tiling_on_sc=False, needs_layout_passes=False),
        scratch_types=[
            pltpu.VMEM((QC, P_PER_W), jnp.float32),
            pltpu.VMEM((P_PER_W, SIZE), jnp.float32),
            pltpu.VMEM((BATCH,), jnp.float32),
            pltpu.VMEM((SIZE,), jnp.float32),
            pltpu.VMEM((3, L), jnp.float32),
            pltpu.VMEM((P_PER_W, QC), jnp.float32),
        ],
    )
    def k(x_hbm, coef_hbm, scal_hbm, knots_hbm, par_hbm, out_hbm,
          xv, cv, sv, kv, pv, ov):
        wid = lax.axis_index("s") * NC + lax.axis_index("c")
        p0 = wid * P_PER_W

        pltpu.sync_copy(coef_hbm.at[pl.ds(p0, P_PER_W), :], cv)
        pltpu.sync_copy(scal_hbm, sv)
        pltpu.sync_copy(knots_hbm, kv)
        pltpu.sync_copy(par_hbm, pv)

        vlo = pv[0]
        vh = pv[1]
        vinv_h = pv[2]
        viota = lax.iota(jnp.int32, L)
        vone = jnp.full((L,), 1.0, jnp.float32)
        kmax = jnp.full((L,), SIZE - 2, jnp.int32)
        kzero = jnp.full((L,), 0, jnp.int32)

        def chunk_body(qc_i, _):
            q0 = qc_i * QC
            pltpu.sync_copy(x_hbm.at[pl.ds(q0, QC), pl.ds(p0, P_PER_W)], xv)

            def qv_body(qv_i, _):
                qrow = viota + qv_i * L
                svec = sv[pl.ds(q0 + qv_i * L, L)]

                @plsc.parallel_loop(0, P_PER_W, unroll=8)
                def p_body(p_i):
                    pfull = jnp.full((L,), p_i, jnp.int32)
                    xvec = plsc.load_gather(xv, [qrow, pfull])
                    u = (xvec - vlo) * vinv_h
                    est = jnp.clip(u.astype(jnp.int32), kzero, kmax)
                    a = plsc.load_gather(kv, [est])
                    b = plsc.load_gather(kv, [est + 1])
                    adj = jnp.where(xvec > b, 1, jnp.where(xvec <= a, -1, 0))
                    left = jnp.clip(est + adj, kzero, kmax)
                    lv = vlo + left.astype(jnp.float32) * vh
                    t = (xvec - lv) * vinv_h
                    cl = plsc.load_gather(cv, [pfull, left])
                    cr = plsc.load_gather(cv, [pfull, left + 1])
                    r = (cl * t + cr * (vone - t)) * svec
                    ov[p_i, pl.ds(qv_i * L, L)] = r

                return 0

            lax.fori_loop(0, NQV, qv_body, 0)
            pltpu.sync_copy(ov, out_hbm.at[pl.ds(p0, P_PER_W), pl.ds(q0, QC)])
            return 0

        lax.fori_loop(0, NCHUNK, chunk_body, 0)

    return k


_sc_spline = _make_sc_kernel()


def kernel(x, coefficients_vect, scaling, knots):
    x2 = x.reshape(BATCH, NUM_ACT)
    coef2 = coefficients_vect.reshape(NUM_ACT, SIZE)
    scal1 = scaling.reshape(NUM_ACT)
    krow = knots[0]
    lo = krow[0]
    h = (krow[SIZE - 1] - krow[0]) / jnp.float32(SIZE - 1)
    inv_h = jnp.float32(1.0) / h
    params = jnp.stack([
        jnp.full((L,), lo, jnp.float32),
        jnp.full((L,), h, jnp.float32),
        jnp.full((L,), inv_h, jnp.float32),
    ])
    out2 = _sc_spline(x2, coef2, scal1, krow, params)
    return out2.reshape(x.shape)


# speedup vs baseline: 13710.4797x; 1.1084x over previous
import functools

import jax
import jax.numpy as jnp
from jax import lax
from jax.experimental import pallas as pl
from jax.experimental.pallas import tpu as pltpu
from jax.experimental.pallas import tpu_sc as plsc

NUM_ACT = 4096
SIZE = 256
BATCH = 4096

NC = 2
NS = 16
L = 16
NW = NC * NS
P_PER_W = NUM_ACT // NW
QC = 128
NCHUNK = BATCH // QC
NPAIR = NCHUNK // 2
NQV = QC // L


def _make_sc_kernel():
    mesh = plsc.VectorSubcoreMesh(core_axis_name="c", subcore_axis_name="s")

    @functools.partial(
        pl.kernel,
        out_type=jax.ShapeDtypeStruct((NUM_ACT, BATCH), jnp.float32),
        mesh=mesh,
        compiler_params=pltpu.CompilerParams(
            use_tc_tiling_on_sc=False, needs_layout_passes=False),
        scratch_types=[
            pltpu.VMEM((QC, P_PER_W), jnp.float32),
            pltpu.VMEM((QC, P_PER_W), jnp.float32),
            pltpu.VMEM((P_PER_W, SIZE), jnp.float32),
            pltpu.VMEM((BATCH,), jnp.float32),
            pltpu.VMEM((SIZE,), jnp.float32),
            pltpu.VMEM((3, L), jnp.float32),
            pltpu.VMEM((P_PER_W, QC), jnp.float32),
            pltpu.VMEM((P_PER_W, QC), jnp.float32),
            pltpu.SemaphoreType.DMA,
            pltpu.SemaphoreType.DMA,
            pltpu.SemaphoreType.DMA,
            pltpu.SemaphoreType.DMA,
        ],
    )
    def k(x_hbm, coef_hbm, scal_hbm, knots_hbm, par_hbm, out_hbm,
          xv0, xv1, cv, sv, kv, pv, ov0, ov1, sin0, sin1, sout0, sout1):
        wid = lax.axis_index("s") * NC + lax.axis_index("c")
        p0 = wid * P_PER_W

        pltpu.sync_copy(coef_hbm.at[pl.ds(p0, P_PER_W), :], cv)
        pltpu.sync_copy(scal_hbm, sv)
        pltpu.sync_copy(knots_hbm, kv)
        pltpu.sync_copy(par_hbm, pv)

        vlo = pv[0]
        vh = pv[1]
        vinv_h = pv[2]
        viota = lax.iota(jnp.int32, L)
        vone = jnp.full((L,), 1.0, jnp.float32)
        kmax = jnp.full((L,), SIZE - 2, jnp.int32)
        kzero = jnp.full((L,), 0, jnp.int32)

        def xsrc(c):
            return x_hbm.at[pl.ds(c * QC, QC), pl.ds(p0, P_PER_W)]

        def odst(c):
            return out_hbm.at[pl.ds(p0, P_PER_W), pl.ds(c * QC, QC)]

        def compute(c, xvb, ovb):
            q0 = c * QC

            def qv_body(qv_i, _):
                qrow = viota + qv_i * L
                svec = sv[pl.ds(q0 + qv_i * L, L)]

                @plsc.parallel_loop(0, P_PER_W, unroll=8)
                def p_body(p_i):
                    pfull = jnp.full((L,), p_i, jnp.int32)
                    xvec = plsc.load_gather(xvb, [qrow, pfull])
                    u = (xvec - vlo) * vinv_h
                    est = jnp.clip(u.astype(jnp.int32), kzero, kmax)
                    a = plsc.load_gather(kv, [est])
                    b = plsc.load_gather(kv, [est + 1])
                    adj = jnp.where(xvec > b, 1, jnp.where(xvec <= a, -1, 0))
                    left = jnp.clip(est + adj, kzero, kmax)
                    lv = vlo + left.astype(jnp.float32) * vh
                    t = (xvec - lv) * vinv_h
                    cl = plsc.load_gather(cv, [pfull, left])
                    cr = plsc.load_gather(cv, [pfull, left + 1])
                    r = (cl * t + cr * (vone - t)) * svec
                    ovb[p_i, pl.ds(qv_i * L, L)] = r

                return 0

            lax.fori_loop(0, NQV, qv_body, 0)

        pltpu.async_copy(xsrc(0), xv0, sin0)

        def pair_body(i, _):
            c0 = 2 * i
            c1 = c0 + 1
            pltpu.async_copy(xsrc(c1), xv1, sin1)
            pltpu.make_async_copy(xsrc(c0), xv0, sin0).wait()

            @pl.when(i > 0)
            def _():
                pltpu.make_async_copy(ov0, odst(c0 - 2), sout0).wait()

            compute(c0, xv0, ov0)
            pltpu.async_copy(ov0, odst(c0), sout0)

            @pl.when(i < NPAIR - 1)
            def _():
                pltpu.async_copy(xsrc(c0 + 2), xv0, sin0)

            pltpu.make_async_copy(xsrc(c1), xv1, sin1).wait()

            @pl.when(i > 0)
            def _():
                pltpu.make_async_copy(ov1, odst(c1 - 2), sout1).wait()

            compute(c1, xv1, ov1)
            pltpu.async_copy(ov1, odst(c1), sout1)
            return 0

        lax.fori_loop(0, NPAIR, pair_body, 0)
        pltpu.make_async_copy(ov0, odst(NCHUNK - 2), sout0).wait()
        pltpu.make_async_copy(ov1, odst(NCHUNK - 1), sout1).wait()

    return k


_sc_spline = _make_sc_kernel()


def kernel(x, coefficients_vect, scaling, knots):
    x2 = x.reshape(BATCH, NUM_ACT)
    coef2 = coefficients_vect.reshape(NUM_ACT, SIZE)
    scal1 = scaling.reshape(NUM_ACT)
    krow = knots[0]
    lo = krow[0]
    h = (krow[SIZE - 1] - krow[0]) / jnp.float32(SIZE - 1)
    inv_h = jnp.float32(1.0) / h
    params = jnp.stack([
        jnp.full((L,), lo, jnp.float32),
        jnp.full((L,), h, jnp.float32),
        jnp.full((L,), inv_h, jnp.float32),
    ])
    out2 = _sc_spline(x2, coef2, scal1, krow, params)
    return out2.reshape(x.shape)


# speedup vs baseline: 14697.0622x; 1.0720x over previous
import functools

import jax
import jax.numpy as jnp
from jax import lax
from jax.experimental import pallas as pl
from jax.experimental.pallas import tpu as pltpu
from jax.experimental.pallas import tpu_sc as plsc

NUM_ACT = 4096
SIZE = 256
BATCH = 4096

NC = 2
NS = 16
L = 16
NW = NC * NS
P_PER_W = NUM_ACT // NW
QC = 128
NCHUNK = BATCH // QC
NPAIR = NCHUNK // 2
NQV = QC // L


def _make_sc_kernel():
    mesh = plsc.VectorSubcoreMesh(core_axis_name="c", subcore_axis_name="s")

    @functools.partial(
        pl.kernel,
        out_type=jax.ShapeDtypeStruct((NUM_ACT, BATCH), jnp.float32),
        mesh=mesh,
        compiler_params=pltpu.CompilerParams(
            use_tc_tiling_on_sc=False, needs_layout_passes=False),
        scratch_types=[
            pltpu.VMEM((QC, P_PER_W), jnp.float32),
            pltpu.VMEM((QC, P_PER_W), jnp.float32),
            pltpu.VMEM((P_PER_W * SIZE,), jnp.float32),
            pltpu.VMEM((BATCH,), jnp.float32),
            pltpu.VMEM((SIZE,), jnp.float32),
            pltpu.VMEM((3, L), jnp.float32),
            pltpu.VMEM((P_PER_W, QC), jnp.float32),
            pltpu.VMEM((P_PER_W, QC), jnp.float32),
            pltpu.SemaphoreType.DMA,
            pltpu.SemaphoreType.DMA,
            pltpu.SemaphoreType.DMA,
            pltpu.SemaphoreType.DMA,
        ],
    )
    def k(x_hbm, coef_hbm, scal_hbm, knots_hbm, par_hbm, out_hbm,
          xv0, xv1, cv, sv, kv, pv, ov0, ov1, sin0, sin1, sout0, sout1):
        wid = lax.axis_index("s") * NC + lax.axis_index("c")
        p0 = wid * P_PER_W

        pltpu.sync_copy(coef_hbm.at[pl.ds(p0 * SIZE, P_PER_W * SIZE)], cv)
        pltpu.sync_copy(scal_hbm, sv)
        pltpu.sync_copy(knots_hbm, kv)
        pltpu.sync_copy(par_hbm, pv)

        vlo = pv[0]
        vinv_h = pv[2]
        vlo_h = vlo * vinv_h
        viota = lax.iota(jnp.int32, L)
        kmax = jnp.full((L,), SIZE - 2, jnp.int32)
        kzero = jnp.full((L,), 0, jnp.int32)

        def xsrc(c):
            return x_hbm.at[pl.ds(c * QC, QC), pl.ds(p0, P_PER_W)]

        def odst(c):
            return out_hbm.at[pl.ds(p0, P_PER_W), pl.ds(c * QC, QC)]

        def compute(c, xvb, ovb):
            q0 = c * QC

            def qv_body(qv_i, _):
                qrow = viota + qv_i * L
                svec = sv[pl.ds(q0 + qv_i * L, L)]

                @plsc.parallel_loop(0, P_PER_W, unroll=16)
                def p_body(p_i):
                    pfull = jnp.full((L,), p_i, jnp.int32)
                    pbase = jnp.full((L,), p_i * SIZE, jnp.int32)
                    xvec = plsc.load_gather(xvb, [qrow, pfull])
                    u = xvec * vinv_h - vlo_h
                    est = jnp.clip(u.astype(jnp.int32), kzero, kmax)
                    a = plsc.load_gather(kv, [est])
                    b = plsc.load_gather(kv, [est + 1])
                    adj = ((xvec > b).astype(jnp.int32)
                           - (xvec <= a).astype(jnp.int32))
                    left = jnp.clip(est + adj, kzero, kmax)
                    t = u - left.astype(jnp.float32)
                    cidx = pbase + left
                    cl = plsc.load_gather(cv, [cidx])
                    cr = plsc.load_gather(cv, [cidx + 1])
                    r = (cr + t * (cl - cr)) * svec
                    ovb[p_i, pl.ds(qv_i * L, L)] = r

                return 0

            lax.fori_loop(0, NQV, qv_body, 0)

        pltpu.async_copy(xsrc(0), xv0, sin0)

        def pair_body(i, _):
            c0 = 2 * i
            c1 = c0 + 1
            pltpu.async_copy(xsrc(c1), xv1, sin1)
            pltpu.make_async_copy(xsrc(c0), xv0, sin0).wait()

            @pl.when(i > 0)
            def _():
                pltpu.make_async_copy(ov0, odst(c0 - 2), sout0).wait()

            compute(c0, xv0, ov0)
            pltpu.async_copy(ov0, odst(c0), sout0)

            @pl.when(i < NPAIR - 1)
            def _():
                pltpu.async_copy(xsrc(c0 + 2), xv0, sin0)

            pltpu.make_async_copy(xsrc(c1), xv1, sin1).wait()

            @pl.when(i > 0)
            def _():
                pltpu.make_async_copy(ov1, odst(c1 - 2), sout1).wait()

            compute(c1, xv1, ov1)
            pltpu.async_copy(ov1, odst(c1), sout1)
            return 0

        lax.fori_loop(0, NPAIR, pair_body, 0)
        pltpu.make_async_copy(ov0, odst(NCHUNK - 2), sout0).wait()
        pltpu.make_async_copy(ov1, odst(NCHUNK - 1), sout1).wait()

    return k


_sc_spline = _make_sc_kernel()


def kernel(x, coefficients_vect, scaling, knots):
    x2 = x.reshape(BATCH, NUM_ACT)
    scal1 = scaling.reshape(NUM_ACT)
    krow = knots[0]
    lo = krow[0]
    h = (krow[SIZE - 1] - krow[0]) / jnp.float32(SIZE - 1)
    inv_h = jnp.float32(1.0) / h
    params = jnp.stack([
        jnp.full((L,), lo, jnp.float32),
        jnp.full((L,), h, jnp.float32),
        jnp.full((L,), inv_h, jnp.float32),
    ])
    out2 = _sc_spline(x2, coefficients_vect, scal1, krow, params)
    return out2.reshape(x.shape)


# speedup vs baseline: 21461.5200x; 1.4603x over previous
import functools

import jax
import jax.numpy as jnp
from jax import lax
from jax.experimental import pallas as pl
from jax.experimental.pallas import tpu as pltpu
from jax.experimental.pallas import tpu_sc as plsc

NUM_ACT = 4096
SIZE = 256
BATCH = 4096

NC = 2
NS = 16
L = 16
NW = NC * NS
P_PER_W = NUM_ACT // NW
NPV = P_PER_W // L
QC = 128
QCP = QC + 1
NCHUNK = BATCH // QC
NPAIR = NCHUNK // 2


def _make_sc_kernel():
    mesh = plsc.VectorSubcoreMesh(core_axis_name="c", subcore_axis_name="s")

    @functools.partial(
        pl.kernel,
        out_type=jax.ShapeDtypeStruct((NUM_ACT, BATCH), jnp.float32),
        mesh=mesh,
        compiler_params=pltpu.CompilerParams(
            use_tc_tiling_on_sc=False, needs_layout_passes=False),
        scratch_types=[
            pltpu.VMEM((QC, P_PER_W), jnp.float32),
            pltpu.VMEM((QC, P_PER_W), jnp.float32),
            pltpu.VMEM((P_PER_W * SIZE,), jnp.float32),
            pltpu.VMEM((BATCH,), jnp.float32),
            pltpu.VMEM((SIZE * L,), jnp.float32),
            pltpu.VMEM((3, L), jnp.float32),
            pltpu.VMEM((P_PER_W, QCP), jnp.float32),
            pltpu.VMEM((P_PER_W, QCP), jnp.float32),
            pltpu.SemaphoreType.DMA,
            pltpu.SemaphoreType.DMA,
            pltpu.SemaphoreType.DMA,
            pltpu.SemaphoreType.DMA,
        ],
    )
    def k(x_hbm, coef_hbm, scal_hbm, knots_hbm, par_hbm, out_hbm,
          xv0, xv1, cv, sv, kv, pv, ov0, ov1, sin0, sin1, sout0, sout1):
        wid = lax.axis_index("s") * NC + lax.axis_index("c")
        p0 = wid * P_PER_W

        pltpu.sync_copy(coef_hbm.at[pl.ds(p0 * SIZE, P_PER_W * SIZE)], cv)
        pltpu.sync_copy(scal_hbm, sv)
        pltpu.sync_copy(knots_hbm, kv)
        pltpu.sync_copy(par_hbm, pv)

        vlo = pv[0]
        vinv_h = pv[2]
        vlo_h = vlo * vinv_h
        viota = lax.iota(jnp.int32, L)
        kmax = jnp.full((L,), SIZE - 2, jnp.int32)
        kzero = jnp.full((L,), 0, jnp.int32)

        def xsrc(c):
            return x_hbm.at[pl.ds(c * QC, QC), pl.ds(p0, P_PER_W)]

        def odst(c):
            return out_hbm.at[pl.ds(p0, P_PER_W), pl.ds(c * QC, QC)]

        def compute(c, xvb, ovb):
            @plsc.parallel_loop(0, QC, unroll=2)
            def q_body(q_i):
                qg = jnp.full((L,), c * QC + q_i, jnp.int32)
                svec = plsc.load_gather(sv, [qg])
                qcol = jnp.full((L,), q_i, jnp.int32)
                for pv_i in range(NPV):
                    xvec = xvb[q_i, pl.ds(pv_i * L, L)]
                    u = xvec * vinv_h - vlo_h
                    est = jnp.clip(u.astype(jnp.int32), kzero, kmax)
                    e16 = est * L + viota
                    a = plsc.load_gather(kv, [e16])
                    b = plsc.load_gather(kv, [e16 + L])
                    adj = ((xvec > b).astype(jnp.int32)
                           - (xvec <= a).astype(jnp.int32))
                    left = jnp.clip(est + adj, kzero, kmax)
                    t = u - left.astype(jnp.float32)
                    cidx = (viota * SIZE + pv_i * (L * SIZE)) + left
                    cl = plsc.load_gather(cv, [cidx])
                    cr = plsc.load_gather(cv, [cidx + 1])
                    r = (cr + t * (cl - cr)) * svec
                    plsc.store_scatter(ovb, [viota + pv_i * L, qcol], r)

        pltpu.async_copy(xsrc(0), xv0, sin0)

        def pair_body(i, _):
            c0 = 2 * i
            c1 = c0 + 1
            pltpu.async_copy(xsrc(c1), xv1, sin1)
            pltpu.make_async_copy(xsrc(c0), xv0, sin0).wait()

            @pl.when(i > 0)
            def _():
                pltpu.make_async_copy(
                    ov0.at[:, pl.ds(0, QC)], odst(c0 - 2), sout0).wait()

            compute(c0, xv0, ov0)
            pltpu.async_copy(ov0.at[:, pl.ds(0, QC)], odst(c0), sout0)

            @pl.when(i < NPAIR - 1)
            def _():
                pltpu.async_copy(xsrc(c0 + 2), xv0, sin0)

            pltpu.make_async_copy(xsrc(c1), xv1, sin1).wait()

            @pl.when(i > 0)
            def _():
                pltpu.make_async_copy(
                    ov1.at[:, pl.ds(0, QC)], odst(c1 - 2), sout1).wait()

            compute(c1, xv1, ov1)
            pltpu.async_copy(ov1.at[:, pl.ds(0, QC)], odst(c1), sout1)
            return 0

        lax.fori_loop(0, NPAIR, pair_body, 0)
        pltpu.make_async_copy(
            ov0.at[:, pl.ds(0, QC)], odst(NCHUNK - 2), sout0).wait()
        pltpu.make_async_copy(
            ov1.at[:, pl.ds(0, QC)], odst(NCHUNK - 1), sout1).wait()

    return k


_sc_spline = _make_sc_kernel()


def kernel(x, coefficients_vect, scaling, knots):
    x2 = x.reshape(BATCH, NUM_ACT)
    scal1 = scaling.reshape(NUM_ACT)
    krow = knots[0]
    krep = jnp.tile(krow[:, None], (1, L)).reshape(-1)
    lo = krow[0]
    h = (krow[SIZE - 1] - krow[0]) / jnp.float32(SIZE - 1)
    inv_h = jnp.float32(1.0) / h
    params = jnp.stack([
        jnp.full((L,), lo, jnp.float32),
        jnp.full((L,), h, jnp.float32),
        jnp.full((L,), inv_h, jnp.float32),
    ])
    out2 = _sc_spline(x2, coefficients_vect, scal1, krep, params)
    return out2.reshape(x.shape)
